# Initial kernel scaffold; baseline (speedup 1.0000x reference)
import jax
import jax.numpy as jnp
from jax.experimental import pallas as pl


def kernel(features, edge_index, W_self1, W_neigh1, b1, W_self2, W_neigh2, b2):
    raise NotImplementedError("write your pallas kernel here")


# speedup vs baseline: 4.3491x; 4.3491x over previous
import functools

import jax
import jax.numpy as jnp
from jax import lax
from jax.experimental import pallas as pl
from jax.experimental.pallas import tpu as pltpu
from jax.experimental.pallas import tpu_sc as plsc

N = 10000
D = 128
L = 16
NC, NS = 2, 16
NW = NC * NS
K = 128
N_PAD = 10240
DW = 128
ROWS_PER_SUB = N_PAD // NS
BR = 512


def _segsum_body(x_hbm, srci_hbm, dsti_hbm, zeros_hbm, agg_hbm,
                 srci_v, dsti_v, rows_v, gsem, acc_sh):
    cid = lax.axis_index("c")
    sid = lax.axis_index("s")
    wid = sid * NC + cid
    ch = srci_v.shape[0]

    pltpu.sync_copy(zeros_hbm, rows_v)
    base = sid * ROWS_PER_SUB
    for t in range(ROWS_PER_SUB // K):
        pltpu.sync_copy(rows_v, acc_sh.at[pl.ds(base + t * K, K)])

    pltpu.sync_copy(srci_hbm.at[wid], srci_v)
    pltpu.sync_copy(dsti_hbm.at[wid], dsti_v)

    plsc.subcore_barrier()

    def _chunk(j, _):
        pltpu.async_copy(x_hbm.at[srci_v.at[j]], rows_v, gsem).wait()
        pltpu.sync_copy(rows_v, acc_sh.at[dsti_v.at[j]], add=True)
        return 0

    lax.fori_loop(0, ch, _chunk, 0)

    plsc.subcore_barrier()

    pltpu.sync_copy(acc_sh.at[pl.ds(base, ROWS_PER_SUB)],
                    agg_hbm.at[cid, pl.ds(base, ROWS_PER_SUB)])


def _deg_body(dsti_hbm, zeros16_hbm, ones16_hbm, deg_hbm, dsti_v, ones_v, deg_sh):
    cid = lax.axis_index("c")
    sid = lax.axis_index("s")
    wid = sid * NC + cid
    ch = dsti_v.shape[0]

    pltpu.sync_copy(zeros16_hbm, ones_v)
    base = sid * ROWS_PER_SUB
    for t in range(ROWS_PER_SUB // K):
        pltpu.sync_copy(ones_v, deg_sh.at[pl.ds(base + t * K, K)])
    pltpu.sync_copy(ones16_hbm, ones_v)

    pltpu.sync_copy(dsti_hbm.at[wid], dsti_v)

    plsc.subcore_barrier()

    def _chunk(j, _):
        pltpu.sync_copy(ones_v, deg_sh.at[dsti_v.at[j]], add=True)
        return 0

    lax.fori_loop(0, ch, _chunk, 0)

    plsc.subcore_barrier()

    pltpu.sync_copy(deg_sh.at[pl.ds(base, ROWS_PER_SUB)],
                    deg_hbm.at[cid, pl.ds(base, ROWS_PER_SUB)])


def _make_segsum(ch):
    mesh = plsc.VectorSubcoreMesh(core_axis_name="c", subcore_axis_name="s",
                                  num_cores=NC, num_subcores=NS)
    return pl.kernel(
        _segsum_body,
        out_type=jax.ShapeDtypeStruct((NC, N_PAD, D), jnp.float32),
        mesh=mesh,
        scratch_types=[
            pltpu.VMEM((ch, K), jnp.int32),
            pltpu.VMEM((ch, K), jnp.int32),
            pltpu.VMEM((K, D), jnp.float32),
            pltpu.SemaphoreType.DMA,
            pltpu.VMEM_SHARED((N_PAD, D), jnp.float32),
        ])


def _make_deg(ch, w=DW):
    mesh = plsc.VectorSubcoreMesh(core_axis_name="c", subcore_axis_name="s",
                                  num_cores=NC, num_subcores=NS)
    return pl.kernel(
        _deg_body,
        out_type=jax.ShapeDtypeStruct((NC, N_PAD, w), jnp.float32),
        mesh=mesh,
        scratch_types=[
            pltpu.VMEM((ch, K), jnp.int32),
            pltpu.VMEM((K, w), jnp.float32),
            pltpu.VMEM_SHARED((N_PAD, w), jnp.float32),
        ])


def _dense_body(relu, x_ref, agg_ref, deg_ref, ws_ref, wn_ref, b_ref, o_ref):
    x = x_ref[...]
    a = agg_ref[0] + agg_ref[1]
    d = deg_ref[0, :, 0:1] + deg_ref[1, :, 0:1]
    inv = 1.0 / jnp.maximum(d, 1.0)
    out = jnp.dot(x, ws_ref[...], preferred_element_type=jnp.float32)
    out += jnp.dot(a, wn_ref[...], preferred_element_type=jnp.float32) * inv
    out += b_ref[...]
    if relu:
        out = jnp.maximum(out, 0.0)
    o_ref[...] = out


def _dense(x_pad, agg, deg, W_self, W_neigh, b, relu):
    grid = (N_PAD // BR,)
    return pl.pallas_call(
        functools.partial(_dense_body, relu),
        grid=grid,
        in_specs=[
            pl.BlockSpec((BR, D), lambda i: (i, 0)),
            pl.BlockSpec((NC, BR, D), lambda i: (0, i, 0)),
            pl.BlockSpec((NC, BR, DW), lambda i: (0, i, 0)),
            pl.BlockSpec((D, D), lambda i: (0, 0)),
            pl.BlockSpec((D, D), lambda i: (0, 0)),
            pl.BlockSpec((1, D), lambda i: (0, 0)),
        ],
        out_specs=pl.BlockSpec((BR, D), lambda i: (i, 0)),
        out_shape=jax.ShapeDtypeStruct((N_PAD, D), jnp.float32),
    )(x_pad, agg, deg, W_self, W_neigh, b.reshape(1, D))


def kernel(features, edge_index, W_self1, W_neigh1, b1, W_self2, W_neigh2, b2):
    e = edge_index.shape[1]
    ch = -(-e // (NW * K))
    e_pad = NW * K * ch
    src = edge_index[0].astype(jnp.int32)
    dst = edge_index[1].astype(jnp.int32)
    src_p = jnp.concatenate(
        [src, jnp.zeros((e_pad - e,), jnp.int32)]).reshape(NW, ch, K)
    dst_p = jnp.concatenate(
        [dst, jnp.full((e_pad - e,), N, jnp.int32)]).reshape(NW, ch, K)
    x1 = jnp.pad(features, ((0, N_PAD - N), (0, 0)))

    zeros = jnp.zeros((K, D), jnp.float32)
    zeros16 = jnp.zeros((K, DW), jnp.float32)
    ones16 = jnp.ones((K, DW), jnp.float32)

    segsum = _make_segsum(ch)
    deg = _make_deg(ch)(dst_p, zeros16, ones16)

    agg1 = segsum(x1, src_p, dst_p, zeros)
    h1 = _dense(x1, agg1, deg, W_self1, W_neigh1, b1, relu=True)
    agg2 = segsum(h1, src_p, dst_p, zeros)
    out = _dense(h1, agg2, deg, W_self2, W_neigh2, b2, relu=False)
    return out[:N]
